# Initial kernel scaffold; baseline (speedup 1.0000x reference)
import jax
import jax.numpy as jnp
from jax.experimental import pallas as pl


def kernel(x, x_edge_index, x_batch, y, y_edge_index, y_batch, emb, l0_W1, l0_b1, l0_W2, l0_b2, l0_eps, l1_W1, l1_b1, l1_W2, l1_b2, l1_eps, enc_W1, enc_b1, enc_W2, enc_b2):
    raise NotImplementedError("write your pallas kernel here")


# speedup vs baseline: 3.1508x; 3.1508x over previous
import functools

import jax
import jax.numpy as jnp
from jax import lax
from jax.experimental import pallas as pl
from jax.experimental.pallas import tpu as pltpu
from jax.experimental.pallas import tpu_sc as plsc

D = 128
NSUB = 16
NCORE = 2


def _emb_gather(idx_all, emb):
    nt = idx_all.shape[0]
    nw = NCORE * NSUB
    rpw = nt // nw
    nch = rpw // 128

    @functools.partial(
        pl.kernel,
        out_type=jax.ShapeDtypeStruct((nt, D), jnp.float32),
        mesh=plsc.VectorSubcoreMesh(core_axis_name="c", subcore_axis_name="s"),
        scratch_types=[
            pltpu.VMEM((128,), jnp.int32),
            pltpu.VMEM((128, D), jnp.float32),
            pltpu.SemaphoreType.DMA,
        ],
    )
    def k(idx_hbm, emb_hbm, out_hbm, idx_v, rows_v, sem):
        wid = lax.axis_index("s") * NCORE + lax.axis_index("c")
        base = wid * rpw
        for i in range(nch):
            off = base + i * 128
            pltpu.sync_copy(idx_hbm.at[pl.ds(off, 128)], idx_v)
            pltpu.async_copy(emb_hbm.at[idx_v], rows_v, sem).wait()
            pltpu.sync_copy(rows_v, out_hbm.at[pl.ds(off, 128)])

    return k(idx_all, emb)


def _aggregate(h, srcs, dsts, npad):
    nt = h.shape[0]
    e = srcs.shape[0] // NCORE
    epw = e // NSUB
    echunk = 80
    while epw % echunk:
        echunk -= 8
    nch = epw // echunk
    rpw = npad // NSUB

    @functools.partial(
        pl.kernel,
        out_type=jax.ShapeDtypeStruct((nt, D), jnp.float32),
        mesh=plsc.VectorSubcoreMesh(core_axis_name="c", subcore_axis_name="s"),
        scratch_types=[
            pltpu.VMEM_SHARED((npad, D), jnp.float32),
            pltpu.VMEM((echunk,), jnp.int32),
            pltpu.VMEM((echunk,), jnp.int32),
            pltpu.VMEM((echunk, D), jnp.float32),
            pltpu.SemaphoreType.DMA,
        ],
    )
    def k(h_hbm, src_hbm, dst_hbm, out_hbm, acc_sp, si_v, di_v, rows_v, sem):
        c = lax.axis_index("c")
        s = lax.axis_index("s")
        pltpu.sync_copy(h_hbm.at[pl.ds(c * npad + s * rpw, rpw)],
                        acc_sp.at[pl.ds(s * rpw, rpw)])
        plsc.subcore_barrier()
        ebase = c * e + s * epw

        def body(i, carry):
            off = ebase + i * echunk
            pltpu.sync_copy(src_hbm.at[pl.ds(off, echunk)], si_v)
            pltpu.sync_copy(dst_hbm.at[pl.ds(off, echunk)], di_v)
            pltpu.async_copy(h_hbm.at[si_v], rows_v, sem).wait()
            pltpu.sync_copy(rows_v, acc_sp.at[di_v], add=True)
            return carry

        lax.fori_loop(0, nch, body, 0)
        plsc.subcore_barrier()
        pltpu.sync_copy(acc_sp.at[pl.ds(s * rpw, rpw)],
                        out_hbm.at[pl.ds(c * npad + s * rpw, rpw)])

    return k(h, srcs, dsts)


def _gin_mlp(acc, h, W1, b1, W2, b2, eps):
    nt = acc.shape[0]
    br = 1024

    def body(acc_ref, h_ref, W1_ref, b1_ref, W2_ref, b2_ref, eps_ref, out_ref):
        t = acc_ref[...] + eps_ref[0, 0] * h_ref[...]
        t = jnp.maximum(
            jnp.dot(t, W1_ref[...], preferred_element_type=jnp.float32)
            + b1_ref[...], 0.0)
        t = jnp.maximum(
            jnp.dot(t, W2_ref[...], preferred_element_type=jnp.float32)
            + b2_ref[...], 0.0)
        out_ref[...] = t

    return pl.pallas_call(
        body,
        grid=(nt // br,),
        in_specs=[
            pl.BlockSpec((br, D), lambda i: (i, 0)),
            pl.BlockSpec((br, D), lambda i: (i, 0)),
            pl.BlockSpec((D, D), lambda i: (0, 0)),
            pl.BlockSpec((1, D), lambda i: (0, 0)),
            pl.BlockSpec((D, D), lambda i: (0, 0)),
            pl.BlockSpec((1, D), lambda i: (0, 0)),
            pl.BlockSpec(memory_space=pltpu.SMEM),
        ],
        out_specs=pl.BlockSpec((br, D), lambda i: (i, 0)),
        out_shape=jax.ShapeDtypeStruct((nt, D), jnp.float32),
    )(acc, h, W1, b1.reshape(1, D), W2, b2.reshape(1, D), eps.reshape(1, 1))


def _pool_encode(h, batch2, g2, W1, b1, W2, b2):
    nt = h.shape[0]

    def body(h_ref, b_ref, W1_ref, b1_ref, W2_ref, b2_ref, out_ref):
        cols = lax.broadcasted_iota(jnp.int32, (nt, g2), 1)
        oh = (b_ref[...] == cols).astype(jnp.float32)
        sums = lax.dot_general(oh, h_ref[...], (((0,), (0,)), ((), ())),
                               preferred_element_type=jnp.float32)
        ones = jnp.ones((nt, 1), jnp.float32)
        counts = lax.dot_general(oh, ones, (((0,), (0,)), ((), ())),
                                 preferred_element_type=jnp.float32)
        mean = sums / jnp.maximum(counts, 1.0)
        t = jnp.maximum(
            jnp.dot(mean, W1_ref[...], preferred_element_type=jnp.float32)
            + b1_ref[...], 0.0)
        out_ref[...] = (
            jnp.dot(t, W2_ref[...], preferred_element_type=jnp.float32)
            + b2_ref[...])

    return pl.pallas_call(
        body,
        out_shape=jax.ShapeDtypeStruct((g2, D), jnp.float32),
    )(h, batch2, W1, b1.reshape(1, D), W2, b2.reshape(1, D))


def kernel(x, x_edge_index, x_batch, y, y_edge_index, y_batch, emb,
           l0_W1, l0_b1, l0_W2, l0_b2, l0_eps,
           l1_W1, l1_b1, l1_W2, l1_b2, l1_eps,
           enc_W1, enc_b1, enc_W2, enc_b2):
    n = x.shape[0]
    g = 64
    npad = -(-n // 2048) * 2048
    nt = 2 * npad
    pad = jnp.zeros((npad - n,), jnp.int32)
    idx_all = jnp.concatenate([x, pad, y, pad])
    srcs = jnp.concatenate([x_edge_index[0], y_edge_index[0] + npad])
    dsts = jnp.concatenate([x_edge_index[1], y_edge_index[1]])
    bpad = jnp.full((npad - n,), 2 * g, jnp.int32)
    batch2 = jnp.concatenate([x_batch, bpad, y_batch + g, bpad]).reshape(nt, 1)

    h = _emb_gather(idx_all, emb)
    acc = _aggregate(h, srcs, dsts, npad)
    h = _gin_mlp(acc, h, l0_W1, l0_b1, l0_W2, l0_b2, l0_eps)
    acc = _aggregate(h, srcs, dsts, npad)
    h = _gin_mlp(acc, h, l1_W1, l1_b1, l1_W2, l1_b2, l1_eps)
    e = _pool_encode(h, batch2, 2 * g, enc_W1, enc_b1, enc_W2, enc_b2)
    return (e[:g], e[g:])
